# baseline (speedup 1.0000x reference)
import functools

import jax
import jax.numpy as jnp
from jax import lax
from jax.experimental import pallas as pl
from jax.experimental.pallas import tpu as pltpu
from jax.experimental.pallas import tpu_sc as plsc

IN_DIM = 1024
HIDDEN = 2048
N_EXPERTS = 8
N_TOKENS = 4096
TILE = 256
NT = N_TOKENS // TILE + N_EXPERTS
PAD = NT * TILE

NW = 32
TOK_W = N_TOKENS // NW
SLOT_W = PAD // NW


def _router_body(x_ref, wsw_ref, bsw_ref,
                 pm_ref, dst_ref, teid_ref, valid_ref,
                 oh_ref, rank_ref):
    x = x_ref[...]
    logits = lax.dot_general(x, wsw_ref[...], (((1,), (1,)), ((), ())),
                             preferred_element_type=jnp.float32)
    logits = logits + bsw_ref[...]
    m = jnp.max(logits, axis=1, keepdims=True)
    ex = jnp.exp(logits - m)
    s = jnp.sum(ex, axis=1, keepdims=True)
    pm_ref[...] = 1.0 / s

    idx = lax.broadcasted_iota(jnp.int32, logits.shape, 1)
    rt = jnp.min(jnp.where(logits == m, idx, N_EXPERTS), axis=1, keepdims=True)
    eids = lax.broadcasted_iota(jnp.int32, (N_TOKENS, N_EXPERTS), 1)
    oh_ref[...] = (rt == eids).astype(jnp.float32)

    BLK = 1024
    r_iota = lax.broadcasted_iota(jnp.int32, (BLK, BLK), 0)
    c_iota = lax.broadcasted_iota(jnp.int32, (BLK, BLK), 1)
    tril = (r_iota >= c_iota).astype(jnp.float32)

    def blk(b, carry):
        ohb = oh_ref[pl.ds(b * BLK, BLK), :]
        cum = lax.dot_general(tril, ohb, (((1,), (0,)), ((), ())),
                              preferred_element_type=jnp.float32) + carry
        rank_ref[pl.ds(b * BLK, BLK), :] = (
            jnp.sum(cum * ohb, axis=1, keepdims=True) - 1.0)
        return cum[BLK - 1:BLK, :]

    counts = lax.fori_loop(0, N_TOKENS // BLK, blk,
                           jnp.zeros((1, N_EXPERTS), jnp.float32))

    ntiles = jnp.floor((counts + (TILE - 1)) / TILE)
    e_r = lax.broadcasted_iota(jnp.int32, (N_EXPERTS, N_EXPERTS), 0)
    e_c = lax.broadcasted_iota(jnp.int32, (N_EXPERTS, N_EXPERTS), 1)
    strict = (e_r < e_c).astype(jnp.float32)
    excl = lax.dot_general(ntiles, strict, (((1,), (0,)), ((), ())),
                           preferred_element_type=jnp.float32)
    start_rows = excl * TILE

    sg = jnp.sum(oh_ref[...] * start_rows, axis=1, keepdims=True)
    dst_ref[...] = (sg + rank_ref[...]).astype(jnp.int32)

    end_tiles = excl + ntiles
    total = jnp.sum(ntiles, axis=1, keepdims=True)
    t_col = lax.broadcasted_iota(jnp.int32, (128, 1), 0).astype(jnp.float32)
    teid = jnp.sum((t_col >= end_tiles).astype(jnp.int32),
                   axis=1, keepdims=True)
    teid_ref[...] = jnp.minimum(teid, N_EXPERTS - 1)
    valid_ref[...] = (t_col < total).astype(jnp.int32)


def _router(xf, W_sw, b_sw):
    return pl.pallas_call(
        _router_body,
        in_specs=[
            pl.BlockSpec((N_TOKENS, IN_DIM), lambda: (0, 0)),
            pl.BlockSpec((N_EXPERTS, IN_DIM), lambda: (0, 0)),
            pl.BlockSpec((1, N_EXPERTS), lambda: (0, 0)),
        ],
        out_specs=[
            pl.BlockSpec((N_TOKENS, 1), lambda: (0, 0)),
            pl.BlockSpec((N_TOKENS, 1), lambda: (0, 0)),
            pl.BlockSpec((128, 1), lambda: (0, 0)),
            pl.BlockSpec((128, 1), lambda: (0, 0)),
        ],
        out_shape=[
            jax.ShapeDtypeStruct((N_TOKENS, 1), jnp.float32),
            jax.ShapeDtypeStruct((N_TOKENS, 1), jnp.int32),
            jax.ShapeDtypeStruct((128, 1), jnp.int32),
            jax.ShapeDtypeStruct((128, 1), jnp.int32),
        ],
        scratch_shapes=[
            pltpu.VMEM((N_TOKENS, N_EXPERTS), jnp.float32),
            pltpu.VMEM((N_TOKENS, 1), jnp.float32),
        ],
    )(xf, W_sw, b_sw.reshape(1, N_EXPERTS))


def _dispatch_body(xf_hbm, dst_hbm, pm_hbm, zero_hbm,
                   xs_hbm, ps_hbm,
                   dstv, gloc, pmv, psv, rows, rows2, sem, sem2, wsem, wsem2):
    wid = lax.axis_index("s") * 2 + lax.axis_index("c")
    sbase = wid * SLOT_W

    pltpu.sync_copy(zero_hbm, gloc)
    pltpu.sync_copy(dst_hbm, dstv)
    pltpu.sync_copy(pm_hbm, pmv)

    def scat(c, _):
        for u in range(4):
            idxs = dstv[pl.ds(c * 64 + u * 16, 16)]
            vals = c * 64 + u * 16 + lax.iota(jnp.int32, 16)
            plsc.store_scatter(gloc, [idxs], vals)
        return _

    lax.fori_loop(0, N_TOKENS // 64, scat, 0)

    CH = 48
    NCH = SLOT_W // CH
    bufs = (rows, rows2)
    gsems = (sem, sem2)
    wsems = (wsem, wsem2)
    gops = [None] * NCH
    wops = [None] * NCH
    for c in range(NCH):
        if c >= 2:
            wops[c - 2].wait()
        gops[c] = pltpu.async_copy(
            xf_hbm.at[gloc.at[pl.ds(sbase + c * CH, CH)]],
            bufs[c % 2], gsems[c % 2])
        if c >= 1:
            gops[c - 1].wait()
            wops[c - 1] = pltpu.async_copy(
                bufs[(c - 1) % 2],
                xs_hbm.at[pl.ds(sbase + (c - 1) * CH, CH)],
                wsems[(c - 1) % 2])
    gops[NCH - 1].wait()
    wops[NCH - 1] = pltpu.async_copy(
        bufs[(NCH - 1) % 2],
        xs_hbm.at[pl.ds(sbase + (NCH - 1) * CH, CH)],
        wsems[(NCH - 1) % 2])

    for c in range(SLOT_W // 16):
        idxs = gloc[pl.ds(sbase + c * 16, 16)]
        psv[pl.ds(c * 16, 16)] = plsc.load_gather(pmv, [idxs])
    pltpu.sync_copy(psv, ps_hbm.at[pl.ds(sbase, SLOT_W)])

    wops[NCH - 2].wait()
    wops[NCH - 1].wait()


def _dispatch(xf, dstf, pmf):
    zero = jnp.arange(PAD, dtype=jnp.int32) % N_TOKENS
    mesh = plsc.VectorSubcoreMesh(core_axis_name="c", subcore_axis_name="s")
    f = pl.kernel(
        _dispatch_body,
        mesh=mesh,
        compiler_params=pltpu.CompilerParams(needs_layout_passes=False),
        out_type=[
            jax.ShapeDtypeStruct((PAD, IN_DIM), jnp.float32),
            jax.ShapeDtypeStruct((PAD,), jnp.float32),
        ],
        scratch_types=[
            pltpu.VMEM((N_TOKENS,), jnp.int32),
            pltpu.VMEM((PAD,), jnp.int32),
            pltpu.VMEM((N_TOKENS,), jnp.float32),
            pltpu.VMEM((SLOT_W,), jnp.float32),
            pltpu.VMEM((48, IN_DIM), jnp.float32),
            pltpu.VMEM((48, IN_DIM), jnp.float32),
            pltpu.SemaphoreType.DMA,
            pltpu.SemaphoreType.DMA,
            pltpu.SemaphoreType.DMA,
            pltpu.SemaphoreType.DMA,
        ],
    )
    return f(xf, dstf, pmf, zero)


def _mlp_body(eid_ref, valid_ref, xs_ref, w1_ref, b1_ref, w2_ref, b2_ref,
              ps_ref, ys_ref, w1b_s, w2b_s):
    t = pl.program_id(0)

    changed = (t == 0) | (eid_ref[t] != eid_ref[jnp.maximum(t - 1, 0)])

    @pl.when(changed)
    def _cast():
        w1b_s[...] = w1_ref[0].astype(jnp.bfloat16)
        w2b_s[...] = w2_ref[0].astype(jnp.bfloat16)

    @pl.when(valid_ref[t] == 1)
    def _go():
        h = lax.dot_general(xs_ref[...].astype(jnp.bfloat16), w1b_s[...],
                            (((1,), (1,)), ((), ())),
                            preferred_element_type=jnp.float32)
        h = jnp.maximum(h + b1_ref[0], 0.0)
        eo = lax.dot_general(h.astype(jnp.bfloat16), w2b_s[...],
                             (((1,), (1,)), ((), ())),
                             preferred_element_type=jnp.float32)
        ys_ref[...] = (eo + b2_ref[0]) * ps_ref[0]


def _grouped_mlp(eid, valid, xs, w1b, b1, w2b, b2, ps):
    grid_spec = pltpu.PrefetchScalarGridSpec(
        num_scalar_prefetch=2,
        grid=(NT,),
        in_specs=[
            pl.BlockSpec((TILE, IN_DIM), lambda t, eid, vld: (t, 0)),
            pl.BlockSpec((1, HIDDEN, IN_DIM), lambda t, eid, vld: (eid[t], 0, 0)),
            pl.BlockSpec((1, 1, HIDDEN), lambda t, eid, vld: (eid[t], 0, 0)),
            pl.BlockSpec((1, IN_DIM, HIDDEN), lambda t, eid, vld: (eid[t], 0, 0)),
            pl.BlockSpec((1, 1, IN_DIM), lambda t, eid, vld: (eid[t], 0, 0)),
            pl.BlockSpec((1, TILE, 1), lambda t, eid, vld: (t, 0, 0)),
        ],
        out_specs=pl.BlockSpec((TILE, IN_DIM), lambda t, eid, vld: (t, 0)),
        scratch_shapes=[
            pltpu.VMEM((HIDDEN, IN_DIM), jnp.bfloat16),
            pltpu.VMEM((IN_DIM, HIDDEN), jnp.bfloat16),
        ],
    )
    return pl.pallas_call(
        _mlp_body,
        grid_spec=grid_spec,
        out_shape=jax.ShapeDtypeStruct((PAD, IN_DIM), jnp.float32),
    )(eid, valid, xs, w1b, b1.reshape(N_EXPERTS, 1, HIDDEN),
      w2b, b2.reshape(N_EXPERTS, 1, IN_DIM), ps.reshape(NT, TILE, 1))


def _combine_body(ys_hbm, dst_hbm, out_hbm,
                  dstv, rows, rows2, sem, sem2, wsem, wsem2):
    wid = lax.axis_index("s") * 2 + lax.axis_index("c")
    base = wid * TOK_W
    pltpu.sync_copy(dst_hbm.at[pl.ds(base, TOK_W)], dstv)
    CH = 32
    NCH = TOK_W // CH
    bufs = (rows, rows2)
    gsems = (sem, sem2)
    wsems = (wsem, wsem2)
    gops = [None] * NCH
    wops = [None] * NCH
    for c in range(NCH):
        if c >= 2:
            wops[c - 2].wait()
        gops[c] = pltpu.async_copy(
            ys_hbm.at[dstv.at[pl.ds(c * CH, CH)]], bufs[c % 2], gsems[c % 2])
        if c >= 1:
            gops[c - 1].wait()
            wops[c - 1] = pltpu.async_copy(
                bufs[(c - 1) % 2], out_hbm.at[pl.ds(base + (c - 1) * CH, CH)],
                wsems[(c - 1) % 2])
    gops[NCH - 1].wait()
    wops[NCH - 1] = pltpu.async_copy(
        bufs[(NCH - 1) % 2], out_hbm.at[pl.ds(base + (NCH - 1) * CH, CH)],
        wsems[(NCH - 1) % 2])
    wops[NCH - 2].wait()
    wops[NCH - 1].wait()


def _combine(ys, dstf):
    mesh = plsc.VectorSubcoreMesh(core_axis_name="c", subcore_axis_name="s")
    f = pl.kernel(
        _combine_body,
        mesh=mesh,
        out_type=jax.ShapeDtypeStruct((N_TOKENS, IN_DIM), jnp.float32),
        scratch_types=[
            pltpu.VMEM((TOK_W,), jnp.int32),
            pltpu.VMEM((32, IN_DIM), jnp.float32),
            pltpu.VMEM((32, IN_DIM), jnp.float32),
            pltpu.SemaphoreType.DMA,
            pltpu.SemaphoreType.DMA,
            pltpu.SemaphoreType.DMA,
            pltpu.SemaphoreType.DMA,
        ],
    )
    return f(ys, dstf)


def kernel(x, W_sw, b_sw, W1, b1, W2, b2):
    B, N, T, C = x.shape
    xf = x.reshape(-1, C)

    pm, dst, teid, valid = _router(xf, W_sw, b_sw)
    pmf = pm.reshape(N_TOKENS)
    dstf = dst.reshape(N_TOKENS)
    eid = teid.reshape(128)[:NT]
    vld = valid.reshape(128)[:NT]

    xs, ps = _dispatch(xf, dstf, pmf)

    ys = _grouped_mlp(eid, vld, xs, W1, b1, W2, b2, ps)

    out = _combine(ys, dstf)
    return out.reshape(B, N, T, C)
